# Initial kernel scaffold; baseline (speedup 1.0000x reference)
import jax
import jax.numpy as jnp
from jax.experimental import pallas as pl


def kernel(x, table):
    raise NotImplementedError("write your pallas kernel here")


# speedup vs baseline: 1.0952x; 1.0952x over previous
import functools

import jax
import jax.numpy as jnp
from jax import lax
from jax.experimental import pallas as pl
from jax.experimental.pallas import tpu as pltpu
from jax.experimental.pallas import tpu_sc as plsc

B = 16384 * 50
D = 32
NW = 32
PER_W = B // NW
CHUNK = 1024
N_CHUNKS = PER_W // CHUNK


def _sc_gather(idx_flat, table):
    mesh = plsc.VectorSubcoreMesh(core_axis_name="c", subcore_axis_name="s")

    @functools.partial(
        pl.kernel,
        mesh=mesh,
        out_type=jax.ShapeDtypeStruct((B, D), jnp.float32),
        compiler_params=pltpu.CompilerParams(use_tc_tiling_on_sc=False),
        scratch_types=[
            pltpu.VMEM((CHUNK,), jnp.int32),
            pltpu.VMEM((CHUNK, D), jnp.float32),
            pltpu.SemaphoreType.DMA,
        ],
    )
    def k(idx_hbm, table_hbm, out_hbm, idx_v, rows_v, sem):
        wid = lax.axis_index("s") * 2 + lax.axis_index("c")
        base = wid * PER_W

        def body(i, carry):
            off = base + i * CHUNK
            pltpu.sync_copy(idx_hbm.at[pl.ds(off, CHUNK)], idx_v)
            pltpu.async_copy(table_hbm.at[idx_v], rows_v, sem).wait()
            pltpu.sync_copy(rows_v, out_hbm.at[pl.ds(off, CHUNK)])
            return carry

        lax.fori_loop(0, N_CHUNKS, body, 0)

    return k(idx_flat, table)


def kernel(x, table):
    idx = x.reshape(-1).astype(jnp.int32)
    out = _sc_gather(idx, table)
    return out.reshape(x.shape + (D,))


# speedup vs baseline: 1.1138x; 1.0171x over previous
import functools

import jax
import jax.numpy as jnp
from jax import lax
from jax.experimental import pallas as pl
from jax.experimental.pallas import tpu as pltpu
from jax.experimental.pallas import tpu_sc as plsc

B = 16384 * 50
D = 32
NW = 32
PER_W = B // NW
CHUNK = 640
N_CHUNKS = PER_W // CHUNK
NBUF = 4


def _sc_gather(idx_flat, table):
    mesh = plsc.VectorSubcoreMesh(core_axis_name="c", subcore_axis_name="s")

    @functools.partial(
        pl.kernel,
        mesh=mesh,
        out_type=jax.ShapeDtypeStruct((B, D), jnp.float32),
        compiler_params=pltpu.CompilerParams(use_tc_tiling_on_sc=False),
        scratch_types=[
            pltpu.VMEM((PER_W,), jnp.int32),
            *[pltpu.VMEM((CHUNK, D), jnp.float32) for _ in range(NBUF)],
            *[pltpu.SemaphoreType.DMA for _ in range(NBUF)],
            *[pltpu.SemaphoreType.DMA for _ in range(NBUF)],
        ],
    )
    def k(idx_hbm, table_hbm, out_hbm, idx_v, *bufs_sems):
        rows = bufs_sems[:NBUF]
        gsem = bufs_sems[NBUF:2 * NBUF]
        ssem = bufs_sems[2 * NBUF:3 * NBUF]

        wid = lax.axis_index("s") * 2 + lax.axis_index("c")
        base = wid * PER_W

        pltpu.sync_copy(idx_hbm.at[pl.ds(base, PER_W)], idx_v)

        def gather_start(c, b):
            pltpu.async_copy(
                table_hbm.at[idx_v.at[pl.ds(c * CHUNK, CHUNK)]], rows[b],
                gsem[b])

        def store_start(c, b):
            pltpu.async_copy(
                rows[b], out_hbm.at[pl.ds(base + c * CHUNK, CHUNK)], ssem[b])

        def drain_gather(b):
            pltpu.make_async_copy(
                table_hbm.at[pl.ds(0, CHUNK)], rows[b], gsem[b]).wait()

        def drain_store(b):
            pltpu.make_async_copy(
                rows[b], out_hbm.at[pl.ds(0, CHUNK)], ssem[b]).wait()

        for b in range(NBUF):
            gather_start(b, b)

        def body(i, carry):
            g = i * NBUF
            for b in range(NBUF):
                c = g + b
                drain_gather(b)
                store_start(c, b)
                drain_store(b)
                gather_start(c + NBUF, b)
            return carry

        lax.fori_loop(0, N_CHUNKS // NBUF - 1, body, 0)

        for b in range(NBUF):
            c = N_CHUNKS - NBUF + b
            drain_gather(b)
            store_start(c, b)
        for b in range(NBUF):
            drain_store(b)

    return k(idx_flat, table)


def kernel(x, table):
    idx = x.reshape(-1).astype(jnp.int32)
    out = _sc_gather(idx, table)
    return out.reshape(x.shape + (D,))


# speedup vs baseline: 1.9457x; 1.7468x over previous
import functools

import jax
import jax.numpy as jnp
from jax import lax
from jax.experimental import pallas as pl
from jax.experimental.pallas import tpu as pltpu
from jax.experimental.pallas import tpu_sc as plsc

C = 50
N = 16384
B = C * N
D = 32
NW = 32
PER_W = B // NW
CHUNK = 512
IPC = N // CHUNK
ITEMS_W = PER_W // CHUNK
NBUF = 5


def _sc_gather(idx_flat, table):
    mesh = plsc.VectorSubcoreMesh(core_axis_name="c", subcore_axis_name="s")

    @functools.partial(
        pl.kernel,
        mesh=mesh,
        out_type=jax.ShapeDtypeStruct((C, N, D), jnp.float32),
        compiler_params=pltpu.CompilerParams(use_tc_tiling_on_sc=False),
        scratch_types=[
            pltpu.VMEM((PER_W,), jnp.int32),
            *[pltpu.VMEM((CHUNK, D), jnp.float32) for _ in range(NBUF)],
            *[pltpu.SemaphoreType.DMA for _ in range(NBUF)],
            *[pltpu.SemaphoreType.DMA for _ in range(NBUF)],
        ],
    )
    def k(idx_hbm, table_hbm, out_hbm, idx_v, *bufs_sems):
        rows = bufs_sems[:NBUF]
        gsem = bufs_sems[NBUF:2 * NBUF]
        ssem = bufs_sems[2 * NBUF:3 * NBUF]

        wid = lax.axis_index("s") * 2 + lax.axis_index("c")
        base = wid * ITEMS_W

        pltpu.sync_copy(idx_hbm.at[pl.ds(base * CHUNK, PER_W)], idx_v)

        def gather_start(j, b):
            pltpu.async_copy(
                table_hbm.at[idx_v.at[pl.ds(j * CHUNK, CHUNK)]], rows[b],
                gsem[b])

        def store_start(j, b):
            t = base + j
            col = t // IPC
            i0 = (t % IPC) * CHUNK
            pltpu.async_copy(
                rows[b], out_hbm.at[col, pl.ds(i0, CHUNK)], ssem[b])

        def drain_gather(b):
            pltpu.make_async_copy(
                table_hbm.at[pl.ds(0, CHUNK)], rows[b], gsem[b]).wait()

        def drain_store(b):
            pltpu.make_async_copy(
                rows[b], out_hbm.at[0, pl.ds(0, CHUNK)], ssem[b]).wait()

        for b in range(NBUF):
            gather_start(b, b)

        def body(r, carry):
            g = r * NBUF
            for b in range(NBUF):
                j = g + b
                drain_gather(b)
                store_start(j, b)
                drain_store(b)
                gather_start(j + NBUF, b)
            return carry

        lax.fori_loop(0, ITEMS_W // NBUF - 1, body, 0)

        for b in range(NBUF):
            drain_gather(b)
            store_start(ITEMS_W - NBUF + b, b)
        for b in range(NBUF):
            drain_store(b)

    return k(idx_flat, table)


def kernel(x, table):
    idx = x.T.reshape(-1).astype(jnp.int32)
    out = _sc_gather(idx, table)
    return jnp.transpose(out, (1, 0, 2))
